# baseline (speedup 1.0000x reference)
import functools

import jax
import jax.numpy as jnp
from jax import lax
from jax.experimental import pallas as pl
from jax.experimental.pallas import tpu as pltpu
from jax.experimental.pallas import tpu_sc as plsc

_TOPK = 5
_NPRED = 5000
_NPAD = 5008
_CHUNKS = _NPAD // 16
_NGT = 100
_GTPAD = 104
_B = 8
_NC, _NS = 2, 16
_NW = _NC * _NS
_GT_BLOCKS = _NW // _B
_COLS = _NGT // _GT_BLOCKS
_EPS = 1e-4


def _sc_match(pb_hbm, gt_hbm, out_hbm,
              px0, py0, px1, py1, pap, g0, g1, g2, g3, mbc, mbi, outb):
    c = lax.axis_index("c")
    s = lax.axis_index("s")
    w = s * _NC + c
    b = w // _GT_BLOCKS
    gc = w % _GT_BLOCKS

    pltpu.sync_copy(pb_hbm.at[b, 0], px0)
    pltpu.sync_copy(pb_hbm.at[b, 1], py0)
    pltpu.sync_copy(pb_hbm.at[b, 2], px1)
    pltpu.sync_copy(pb_hbm.at[b, 3], py1)
    pltpu.sync_copy(gt_hbm.at[b, 0], g0)
    pltpu.sync_copy(gt_hbm.at[b, 1], g1)
    pltpu.sync_copy(gt_hbm.at[b, 2], g2)
    pltpu.sync_copy(gt_hbm.at[b, 3], g3)

    lane = lax.iota(jnp.int32, 16)
    inf = float("inf")

    def area_body(i, carry):
        o = i * 16
        pap[pl.ds(o, 16)] = ((px1[pl.ds(o, 16)] - px0[pl.ds(o, 16)])
                             * (py1[pl.ds(o, 16)] - py0[pl.ds(o, 16)]))
        return carry

    lax.fori_loop(0, _CHUNKS, area_body, 0)

    def col_body(col, carry):
        j = gc * _COLS + col
        jv = jnp.full((16,), j, jnp.int32)
        gx0 = plsc.load_gather(g0, [jv])
        gy0 = plsc.load_gather(g1, [jv])
        gx1 = plsc.load_gather(g2, [jv])
        gy1 = plsc.load_gather(g3, [jv])
        area_g = (gx1 - gx0) * (gy1 - gy0)
        inv_ag = 1.0 / area_g
        inf16 = jnp.full((16,), inf, jnp.float32)

        def chunk_body(ch, state):
            run_c, run_i, t4e = state
            o = ch * 16
            x0 = px0[pl.ds(o, 16)]
            y0 = py0[pl.ds(o, 16)]
            x1 = px1[pl.ds(o, 16)]
            y1 = py1[pl.ds(o, 16)]
            ap = pap[pl.ds(o, 16)]
            l1 = (jnp.abs(x0 - gx0) + jnp.abs(y0 - gy0)
                  + jnp.abs(x1 - gx1) + jnp.abs(y1 - gy1))
            bound = l1 + jnp.maximum(1.0 - ap * inv_ag, 0.0)
            hit = jnp.any(bound < t4e)

            def do_update(st):
                run_c, run_i, _ = st
                ltx = jnp.maximum(x0, gx0)
                lty = jnp.maximum(y0, gy0)
                rbx = jnp.minimum(x1, gx1)
                rby = jnp.minimum(y1, gy1)
                whx = jnp.maximum(rbx - ltx, 0.0)
                why = jnp.maximum(rby - lty, 0.0)
                inter = whx * why
                union = ap + area_g - inter
                iou = inter / union
                cxw = jnp.maximum(x1, gx1) - jnp.minimum(x0, gx0)
                cyw = jnp.maximum(y1, gy1) - jnp.minimum(y0, gy0)
                area_c = jnp.maximum(cxw, 0.0) * jnp.maximum(cyw, 0.0)
                giou = iou - (area_c - union) / area_c
                cost = 1.0 * l1 + 1.0 * (1.0 - giou)
                pidx = o + lane
                cost = jnp.where(pidx < _NPRED, cost, inf)
                skey, sval = plsc.sort_key_val(cost, pidx)
                mbc[pl.ds(0, 16)] = run_c
                mbi[pl.ds(0, 16)] = run_i
                mbc[pl.ds(16, 16)] = skey
                mbi[pl.ds(16, 16)] = sval
                gsel = jnp.where(lane < _TOPK, lane, lane + 11)
                mk = plsc.load_gather(mbc, [gsel])
                mv = plsc.load_gather(mbi, [gsel])
                nrun_c, nrun_i = plsc.sort_key_val(mk, mv)
                mbc[pl.ds(0, 16)] = nrun_c
                k4 = jnp.full((16,), _TOPK - 1, jnp.int32)
                nt4e = plsc.load_gather(mbc, [k4]) + _EPS
                return nrun_c, nrun_i, nt4e

            return lax.cond(hit, do_update, lambda st: st,
                            (run_c, run_i, t4e))

        run_c, run_i, t4e = lax.fori_loop(
            0, _CHUNKS, chunk_body,
            (inf16, jnp.zeros((16,), jnp.int32), inf16))
        plsc.store_scatter(outb, [col * _TOPK + lane], run_i,
                           mask=lane < _TOPK)
        return carry

    lax.fori_loop(0, _COLS, col_body, 0)
    pltpu.sync_copy(outb, out_hbm.at[w])


def kernel(pred_box, pred_obj, gt_box, gt_obj):
    del pred_obj, gt_obj
    B, N, _ = pred_box.shape
    M = gt_box.shape[1]
    pb_sc = jnp.pad(pred_box.transpose(0, 2, 1),
                    ((0, 0), (0, 0), (0, _NPAD - N)))
    gt_sc = jnp.pad(gt_box.transpose(0, 2, 1),
                    ((0, 0), (0, 0), (0, _GTPAD - M)))

    run = pl.kernel(
        _sc_match,
        out_type=jax.ShapeDtypeStruct((_NW, 128), jnp.int32),
        mesh=plsc.VectorSubcoreMesh(core_axis_name="c", subcore_axis_name="s",
                                    num_cores=_NC, num_subcores=_NS),
        scratch_types=(
            [pltpu.VMEM((_NPAD,), jnp.float32)] * 5
            + [pltpu.VMEM((_GTPAD,), jnp.float32)] * 4
            + [pltpu.VMEM((32,), jnp.float32),
               pltpu.VMEM((32,), jnp.int32),
               pltpu.VMEM((128,), jnp.int32)]
        ),
        compiler_params=pltpu.CompilerParams(needs_layout_passes=False),
    )
    out = run(pb_sc, gt_sc)
    matched_pred = out[:, :_COLS * _TOPK].reshape(B, M * _TOPK)
    matched_gt = jnp.broadcast_to(
        jnp.repeat(jnp.arange(M, dtype=jnp.int32), _TOPK), (B, M * _TOPK))
    return matched_pred, matched_gt


# speedup vs baseline: 1.0388x; 1.0388x over previous
import functools

import jax
import jax.numpy as jnp
from jax import lax
from jax.experimental import pallas as pl
from jax.experimental.pallas import tpu as pltpu
from jax.experimental.pallas import tpu_sc as plsc

_TOPK = 5
_NPRED = 5000
_NPAD = 5024
_CHUNKS = 313
_NGT = 100
_GTPAD = 104
_B = 8
_NC, _NS = 2, 16
_NW = _NC * _NS
_GT_BLOCKS = _NW // _B
_COLS = _NGT // _GT_BLOCKS
_EPS = 1e-4


def _sc_match(pb_hbm, gt_hbm, out_hbm,
              px0, py0, px1, py1, pap, g0, g1, g2, g3, mbc, mbi, outb):
    c = lax.axis_index("c")
    s = lax.axis_index("s")
    w = s * _NC + c
    b = w // _GT_BLOCKS
    gc = w % _GT_BLOCKS

    pltpu.sync_copy(pb_hbm.at[b, 0], px0)
    pltpu.sync_copy(pb_hbm.at[b, 1], py0)
    pltpu.sync_copy(pb_hbm.at[b, 2], px1)
    pltpu.sync_copy(pb_hbm.at[b, 3], py1)
    pltpu.sync_copy(gt_hbm.at[b, 0], g0)
    pltpu.sync_copy(gt_hbm.at[b, 1], g1)
    pltpu.sync_copy(gt_hbm.at[b, 2], g2)
    pltpu.sync_copy(gt_hbm.at[b, 3], g3)

    lane = lax.iota(jnp.int32, 16)
    inf = float("inf")

    def area_body(i, carry):
        o = i * 16
        pap[pl.ds(o, 16)] = ((px1[pl.ds(o, 16)] - px0[pl.ds(o, 16)])
                             * (py1[pl.ds(o, 16)] - py0[pl.ds(o, 16)]))
        return carry

    lax.fori_loop(0, _NPAD // 16, area_body, 0)

    def col_body(col, carry):
        j = gc * _COLS + col
        jv = jnp.full((16,), j, jnp.int32)
        gx0 = plsc.load_gather(g0, [jv])
        gy0 = plsc.load_gather(g1, [jv])
        gx1 = plsc.load_gather(g2, [jv])
        gy1 = plsc.load_gather(g3, [jv])
        area_g = (gx1 - gx0) * (gy1 - gy0)
        inv_ag = 1.0 / area_g
        inf16 = jnp.full((16,), inf, jnp.float32)

        def bound_of(ch, t4e):
            o = ch * 16
            l1 = (jnp.abs(px0[pl.ds(o, 16)] - gx0)
                  + jnp.abs(py0[pl.ds(o, 16)] - gy0)
                  + jnp.abs(px1[pl.ds(o, 16)] - gx1)
                  + jnp.abs(py1[pl.ds(o, 16)] - gy1))
            bound = l1 + jnp.maximum(1.0 - pap[pl.ds(o, 16)] * inv_ag, 0.0)
            return jnp.any(bound < t4e)

        def outer_cond(st):
            return st[0] < _CHUNKS

        def outer_body(st):
            ch0, excl, run_c, run_i, t4e = st

            def skip_cond(ch):
                return jnp.logical_and(ch < _CHUNKS,
                                       jnp.logical_not(bound_of(ch, t4e)))

            ch = lax.while_loop(skip_cond, lambda ch: ch + 1, ch0)

            o = ch * 16
            x0 = px0[pl.ds(o, 16)]
            y0 = py0[pl.ds(o, 16)]
            x1 = px1[pl.ds(o, 16)]
            y1 = py1[pl.ds(o, 16)]
            ap = pap[pl.ds(o, 16)]
            l1 = (jnp.abs(x0 - gx0) + jnp.abs(y0 - gy0)
                  + jnp.abs(x1 - gx1) + jnp.abs(y1 - gy1))
            ltx = jnp.maximum(x0, gx0)
            lty = jnp.maximum(y0, gy0)
            rbx = jnp.minimum(x1, gx1)
            rby = jnp.minimum(y1, gy1)
            whx = jnp.maximum(rbx - ltx, 0.0)
            why = jnp.maximum(rby - lty, 0.0)
            inter = whx * why
            union = ap + area_g - inter
            iou = inter / union
            cxw = jnp.maximum(x1, gx1) - jnp.minimum(x0, gx0)
            cyw = jnp.maximum(y1, gy1) - jnp.minimum(y0, gy0)
            area_c = jnp.maximum(cxw, 0.0) * jnp.maximum(cyw, 0.0)
            giou = iou - (area_c - union) / area_c
            cost = 1.0 * l1 + 1.0 * (1.0 - giou)
            pidx = o + lane
            cost = jnp.where(pidx < _NPRED, cost, inf)
            hits = jnp.logical_and(cost < t4e, pidx > excl)
            mbc[pl.ds(0, 16)] = run_c
            mbi[pl.ds(0, 16)] = run_i
            plsc.store_compressed(mbc.at[pl.ds(_TOPK, 16)], cost, mask=hits)
            plsc.store_compressed(mbi.at[pl.ds(_TOPK, 16)], pidx, mask=hits)
            nrun_c, nrun_i = plsc.sort_key_val(mbc[pl.ds(0, 16)],
                                               mbi[pl.ds(0, 16)])
            mbc[pl.ds(0, 16)] = nrun_c
            k4 = jnp.full((16,), _TOPK - 1, jnp.int32)
            nt4e = plsc.load_gather(mbc, [k4]) + _EPS
            nhits = jnp.sum(hits.astype(jnp.int32))
            done = nhits <= 16 - _TOPK
            ch_next = jnp.where(done, ch + 1, ch)
            k15 = jnp.full((16,), 15, jnp.int32)
            last_merged = plsc.load_gather(mbi, [k15])
            nexcl = jnp.where(done, jnp.full((16,), -1, jnp.int32),
                              last_merged)
            return ch_next, nexcl, nrun_c, nrun_i, nt4e

        _, _, run_c, run_i, t4e = lax.while_loop(
            outer_cond, outer_body,
            (jnp.int32(0), jnp.full((16,), -1, jnp.int32),
             inf16, jnp.zeros((16,), jnp.int32), inf16))
        plsc.store_scatter(outb, [col * _TOPK + lane], run_i,
                           mask=lane < _TOPK)
        return carry

    lax.fori_loop(0, _COLS, col_body, 0)
    pltpu.sync_copy(outb, out_hbm.at[w])


def kernel(pred_box, pred_obj, gt_box, gt_obj):
    del pred_obj, gt_obj
    B, N, _ = pred_box.shape
    M = gt_box.shape[1]
    pb_sc = jnp.pad(pred_box.transpose(0, 2, 1),
                    ((0, 0), (0, 0), (0, _NPAD - N)))
    gt_sc = jnp.pad(gt_box.transpose(0, 2, 1),
                    ((0, 0), (0, 0), (0, _GTPAD - M)))

    run = pl.kernel(
        _sc_match,
        out_type=jax.ShapeDtypeStruct((_NW, 128), jnp.int32),
        mesh=plsc.VectorSubcoreMesh(core_axis_name="c", subcore_axis_name="s",
                                    num_cores=_NC, num_subcores=_NS),
        scratch_types=(
            [pltpu.VMEM((_NPAD,), jnp.float32)] * 5
            + [pltpu.VMEM((_GTPAD,), jnp.float32)] * 4
            + [pltpu.VMEM((32,), jnp.float32),
               pltpu.VMEM((32,), jnp.int32),
               pltpu.VMEM((128,), jnp.int32)]
        ),
        compiler_params=pltpu.CompilerParams(needs_layout_passes=False),
    )
    out = run(pb_sc, gt_sc)
    matched_pred = out[:, :_COLS * _TOPK].reshape(B, M * _TOPK)
    matched_gt = jnp.broadcast_to(
        jnp.repeat(jnp.arange(M, dtype=jnp.int32), _TOPK), (B, M * _TOPK))
    return matched_pred, matched_gt


# speedup vs baseline: 2.0265x; 1.9509x over previous
import functools

import jax
import jax.numpy as jnp
from jax import lax
from jax.experimental import pallas as pl
from jax.experimental.pallas import tpu as pltpu
from jax.experimental.pallas import tpu_sc as plsc

_TOPK = 5
_NPRED = 5000
_NPAD = 5024
_CHUNKS = 313
_NGT = 100
_GTPAD = 104
_B = 8
_NC, _NS = 2, 16
_NW = _NC * _NS
_GT_BLOCKS = _NW // _B
_COLS = _NGT // _GT_BLOCKS


def _sc_match(pb_hbm, gt_hbm, out_hbm,
              px0, py0, px1, py1, pap, g0, g1, g2, g3,
              cbuf, candc, candi, mbc, mbi, outb):
    c = lax.axis_index("c")
    s = lax.axis_index("s")
    w = s * _NC + c
    b = w // _GT_BLOCKS
    gc = w % _GT_BLOCKS

    pltpu.sync_copy(pb_hbm.at[b, 0], px0)
    pltpu.sync_copy(pb_hbm.at[b, 1], py0)
    pltpu.sync_copy(pb_hbm.at[b, 2], px1)
    pltpu.sync_copy(pb_hbm.at[b, 3], py1)
    pltpu.sync_copy(gt_hbm.at[b, 0], g0)
    pltpu.sync_copy(gt_hbm.at[b, 1], g1)
    pltpu.sync_copy(gt_hbm.at[b, 2], g2)
    pltpu.sync_copy(gt_hbm.at[b, 3], g3)

    lane = lax.iota(jnp.int32, 16)
    inf = float("inf")

    def area_body(i, carry):
        o = i * 16
        pap[pl.ds(o, 16)] = ((px1[pl.ds(o, 16)] - px0[pl.ds(o, 16)])
                             * (py1[pl.ds(o, 16)] - py0[pl.ds(o, 16)]))
        return carry

    lax.fori_loop(0, _NPAD // 16, area_body, 0)

    def col_body(col, carry):
        j = gc * _COLS + col
        jv = jnp.full((16,), j, jnp.int32)
        gx0 = plsc.load_gather(g0, [jv])
        gy0 = plsc.load_gather(g1, [jv])
        gx1 = plsc.load_gather(g2, [jv])
        gy1 = plsc.load_gather(g3, [jv])
        area_g = (gx1 - gx0) * (gy1 - gy0)
        inf16 = jnp.full((16,), inf, jnp.float32)

        def pass_a(ch, runmin):
            o = ch * 16
            x0 = px0[pl.ds(o, 16)]
            y0 = py0[pl.ds(o, 16)]
            x1 = px1[pl.ds(o, 16)]
            y1 = py1[pl.ds(o, 16)]
            ap = pap[pl.ds(o, 16)]
            l1 = (jnp.abs(x0 - gx0) + jnp.abs(y0 - gy0)
                  + jnp.abs(x1 - gx1) + jnp.abs(y1 - gy1))
            ltx = jnp.maximum(x0, gx0)
            lty = jnp.maximum(y0, gy0)
            rbx = jnp.minimum(x1, gx1)
            rby = jnp.minimum(y1, gy1)
            whx = jnp.maximum(rbx - ltx, 0.0)
            why = jnp.maximum(rby - lty, 0.0)
            inter = whx * why
            union = ap + area_g - inter
            iou = inter / union
            cxw = jnp.maximum(x1, gx1) - jnp.minimum(x0, gx0)
            cyw = jnp.maximum(y1, gy1) - jnp.minimum(y0, gy0)
            area_c = jnp.maximum(cxw, 0.0) * jnp.maximum(cyw, 0.0)
            giou = iou - (area_c - union) / area_c
            cost = 1.0 * l1 + 1.0 * (1.0 - giou)
            pidx = o + lane
            cost = jnp.where(pidx < _NPRED, cost, inf)
            cbuf[pl.ds(o, 16)] = cost
            return jnp.minimum(runmin, cost)

        runmin = lax.fori_loop(0, _CHUNKS, pass_a, inf16)

        smin, _ = plsc.sort_key_val(runmin, lane)
        mbc[pl.ds(0, 16)] = smin
        k4 = jnp.full((16,), _TOPK - 1, jnp.int32)
        tau = plsc.load_gather(mbc, [k4])

        def pass_b(ch, cnt):
            o = ch * 16
            cval = cbuf[pl.ds(o, 16)]
            m = cval <= tau
            pref = plsc.cumsum(m.astype(jnp.int32))
            tgt = cnt + pref - 1
            plsc.store_scatter(candc, [tgt], cval, mask=m)
            plsc.store_scatter(candi, [tgt], o + lane, mask=m)
            return cnt + plsc.all_reduce_population_count(m)

        cnt = lax.fori_loop(0, _CHUNKS, pass_b,
                            jnp.zeros((16,), jnp.int32))
        cnt_s = jnp.max(cnt)
        ngroups = (cnt_s + 15) // 16

        def pass_c(g, st):
            run_c, run_i = st
            o = g * 16
            ordn = o + lane
            cval = candc[pl.ds(o, 16)]
            ival = candi[pl.ds(o, 16)]
            cval = jnp.where(ordn < cnt, cval, inf)
            skey, sval = plsc.sort_key_val(cval, ival)
            mbc[pl.ds(0, 16)] = run_c
            mbi[pl.ds(0, 16)] = run_i
            mbc[pl.ds(16, 16)] = skey
            mbi[pl.ds(16, 16)] = sval
            gsel = jnp.where(lane < _TOPK, lane, lane + 11)
            mk = plsc.load_gather(mbc, [gsel])
            mv = plsc.load_gather(mbi, [gsel])
            nk, nv = plsc.sort_key_val(mk, mv)
            return (nk, nv)

        run_c, run_i = lax.fori_loop(
            0, ngroups, pass_c, (inf16, jnp.zeros((16,), jnp.int32)))

        plsc.store_scatter(outb, [col * _TOPK + lane], run_i,
                           mask=lane < _TOPK)
        return carry

    lax.fori_loop(0, _COLS, col_body, 0)
    pltpu.sync_copy(outb, out_hbm.at[w])


def kernel(pred_box, pred_obj, gt_box, gt_obj):
    del pred_obj, gt_obj
    B, N, _ = pred_box.shape
    M = gt_box.shape[1]
    pb_sc = jnp.pad(pred_box.transpose(0, 2, 1),
                    ((0, 0), (0, 0), (0, _NPAD - N)))
    gt_sc = jnp.pad(gt_box.transpose(0, 2, 1),
                    ((0, 0), (0, 0), (0, _GTPAD - M)))

    run = pl.kernel(
        _sc_match,
        out_type=jax.ShapeDtypeStruct((_NW, 128), jnp.int32),
        mesh=plsc.VectorSubcoreMesh(core_axis_name="c", subcore_axis_name="s",
                                    num_cores=_NC, num_subcores=_NS),
        scratch_types=(
            [pltpu.VMEM((_NPAD,), jnp.float32)] * 5
            + [pltpu.VMEM((_GTPAD,), jnp.float32)] * 4
            + [pltpu.VMEM((_NPAD,), jnp.float32),
               pltpu.VMEM((_NPAD,), jnp.float32),
               pltpu.VMEM((_NPAD,), jnp.int32),
               pltpu.VMEM((32,), jnp.float32),
               pltpu.VMEM((32,), jnp.int32),
               pltpu.VMEM((128,), jnp.int32)]
        ),
        compiler_params=pltpu.CompilerParams(needs_layout_passes=False),
    )
    out = run(pb_sc, gt_sc)
    matched_pred = out[:, :_COLS * _TOPK].reshape(B, M * _TOPK)
    matched_gt = jnp.broadcast_to(
        jnp.repeat(jnp.arange(M, dtype=jnp.int32), _TOPK), (B, M * _TOPK))
    return matched_pred, matched_gt


# speedup vs baseline: 3.0801x; 1.5199x over previous
import functools

import jax
import jax.numpy as jnp
from jax import lax
from jax.experimental import pallas as pl
from jax.experimental.pallas import tpu as pltpu
from jax.experimental.pallas import tpu_sc as plsc

_TOPK = 5
_NPRED = 5000
_NPAD = 5024
_CHUNKS = 314
_NGT = 100
_GTPAD = 104
_B = 8
_NC, _NS = 2, 16
_NW = _NC * _NS
_GT_BLOCKS = _NW // _B
_COLS = _NGT // _GT_BLOCKS


def _sc_match(pb_hbm, gt_hbm, out_hbm,
              px0, py0, px1, py1, pap, g0, g1, g2, g3,
              cbuf, candc, candi, mbc, mbi, outb):
    c = lax.axis_index("c")
    s = lax.axis_index("s")
    w = s * _NC + c
    b = w // _GT_BLOCKS
    gc = w % _GT_BLOCKS

    pltpu.sync_copy(pb_hbm.at[b, 0], px0)
    pltpu.sync_copy(pb_hbm.at[b, 1], py0)
    pltpu.sync_copy(pb_hbm.at[b, 2], px1)
    pltpu.sync_copy(pb_hbm.at[b, 3], py1)
    pltpu.sync_copy(gt_hbm.at[b, 0], g0)
    pltpu.sync_copy(gt_hbm.at[b, 1], g1)
    pltpu.sync_copy(gt_hbm.at[b, 2], g2)
    pltpu.sync_copy(gt_hbm.at[b, 3], g3)

    lane = lax.iota(jnp.int32, 16)
    inf = float("inf")

    def area_body(i, carry):
        o = i * 16
        pap[pl.ds(o, 16)] = ((px1[pl.ds(o, 16)] - px0[pl.ds(o, 16)])
                             * (py1[pl.ds(o, 16)] - py0[pl.ds(o, 16)]))
        return carry

    lax.fori_loop(0, _NPAD // 16, area_body, 0)

    def col_body(col, carry):
        j = gc * _COLS + col
        jv = jnp.full((16,), j, jnp.int32)
        gx0 = plsc.load_gather(g0, [jv])
        gy0 = plsc.load_gather(g1, [jv])
        gx1 = plsc.load_gather(g2, [jv])
        gy1 = plsc.load_gather(g3, [jv])
        area_g = (gx1 - gx0) * (gy1 - gy0)
        inf16 = jnp.full((16,), inf, jnp.float32)

        def cost_of(o):
            x0 = px0[pl.ds(o, 16)]
            y0 = py0[pl.ds(o, 16)]
            x1 = px1[pl.ds(o, 16)]
            y1 = py1[pl.ds(o, 16)]
            ap = pap[pl.ds(o, 16)]
            l1 = (jnp.abs(x0 - gx0) + jnp.abs(y0 - gy0)
                  + jnp.abs(x1 - gx1) + jnp.abs(y1 - gy1))
            ltx = jnp.maximum(x0, gx0)
            lty = jnp.maximum(y0, gy0)
            rbx = jnp.minimum(x1, gx1)
            rby = jnp.minimum(y1, gy1)
            whx = jnp.maximum(rbx - ltx, 0.0)
            why = jnp.maximum(rby - lty, 0.0)
            inter = whx * why
            union = ap + area_g - inter
            iou = inter / union
            cxw = jnp.maximum(x1, gx1) - jnp.minimum(x0, gx0)
            cyw = jnp.maximum(y1, gy1) - jnp.minimum(y0, gy0)
            area_c = jnp.maximum(cxw, 0.0) * jnp.maximum(cyw, 0.0)
            giou = iou - (area_c - union) / area_c
            return 1.0 * l1 + 1.0 * (1.0 - giou)

        def pass_a(ch, runmin):
            o = ch * 32
            c0 = cost_of(o)
            c1 = cost_of(o + 16)
            cbuf[pl.ds(o, 16)] = c0
            cbuf[pl.ds(o + 16, 16)] = c1
            return jnp.minimum(runmin, jnp.minimum(c0, c1))

        runmin = lax.fori_loop(0, _CHUNKS // 2, pass_a, inf16)

        smin, _ = plsc.sort_key_val(runmin, lane)
        mbc[pl.ds(0, 16)] = smin
        k4 = jnp.full((16,), _TOPK - 1, jnp.int32)
        tau = plsc.load_gather(mbc, [k4])

        def append(o, cnt):
            cval = cbuf[pl.ds(o, 16)]
            m = cval <= tau
            pref = plsc.cumsum(m.astype(jnp.int32))
            tgt = cnt + pref - 1
            plsc.store_scatter(candc, [tgt], cval, mask=m)
            plsc.store_scatter(candi, [tgt], o + lane, mask=m)
            return cnt + plsc.all_reduce_population_count(m)

        def pass_b(ch, cnt):
            o = ch * 32
            cnt = append(o, cnt)
            return append(o + 16, cnt)

        cnt = lax.fori_loop(0, _CHUNKS // 2, pass_b,
                            jnp.zeros((16,), jnp.int32))
        cnt_s = jnp.max(cnt)
        ngroups = (cnt_s + 15) // 16

        def pass_c(g, st):
            run_c, run_i = st
            o = g * 16
            ordn = o + lane
            cval = candc[pl.ds(o, 16)]
            ival = candi[pl.ds(o, 16)]
            cval = jnp.where(ordn < cnt, cval, inf)
            skey, sval = plsc.sort_key_val(cval, ival)
            mbc[pl.ds(0, 16)] = run_c
            mbi[pl.ds(0, 16)] = run_i
            mbc[pl.ds(16, 16)] = skey
            mbi[pl.ds(16, 16)] = sval
            gsel = jnp.where(lane < _TOPK, lane, lane + 11)
            mk = plsc.load_gather(mbc, [gsel])
            mv = plsc.load_gather(mbi, [gsel])
            nk, nv = plsc.sort_key_val(mk, mv)
            return (nk, nv)

        run_c, run_i = lax.fori_loop(
            0, ngroups, pass_c, (inf16, jnp.zeros((16,), jnp.int32)))

        plsc.store_scatter(outb, [col * _TOPK + lane], run_i,
                           mask=lane < _TOPK)
        return carry

    lax.fori_loop(0, _COLS, col_body, 0)
    pltpu.sync_copy(outb, out_hbm.at[w])


def kernel(pred_box, pred_obj, gt_box, gt_obj):
    del pred_obj, gt_obj
    B, N, _ = pred_box.shape
    M = gt_box.shape[1]
    poison = jnp.array([1e6, 1e6, 1e6 + 1.0, 1e6 + 1.0],
                       dtype=jnp.float32).reshape(1, 4, 1)
    pb_sc = jnp.concatenate(
        [pred_box.transpose(0, 2, 1),
         jnp.broadcast_to(poison, (B, 4, _NPAD - N))], axis=2)
    gt_sc = jnp.pad(gt_box.transpose(0, 2, 1),
                    ((0, 0), (0, 0), (0, _GTPAD - M)))

    run = pl.kernel(
        _sc_match,
        out_type=jax.ShapeDtypeStruct((_NW, 128), jnp.int32),
        mesh=plsc.VectorSubcoreMesh(core_axis_name="c", subcore_axis_name="s",
                                    num_cores=_NC, num_subcores=_NS),
        scratch_types=(
            [pltpu.VMEM((_NPAD,), jnp.float32)] * 5
            + [pltpu.VMEM((_GTPAD,), jnp.float32)] * 4
            + [pltpu.VMEM((_NPAD,), jnp.float32),
               pltpu.VMEM((_NPAD,), jnp.float32),
               pltpu.VMEM((_NPAD,), jnp.int32),
               pltpu.VMEM((32,), jnp.float32),
               pltpu.VMEM((32,), jnp.int32),
               pltpu.VMEM((128,), jnp.int32)]
        ),
        compiler_params=pltpu.CompilerParams(needs_layout_passes=False),
    )
    out = run(pb_sc, gt_sc)
    matched_pred = out[:, :_COLS * _TOPK].reshape(B, M * _TOPK)
    matched_gt = jnp.broadcast_to(
        jnp.repeat(jnp.arange(M, dtype=jnp.int32), _TOPK), (B, M * _TOPK))
    return matched_pred, matched_gt


# speedup vs baseline: 3.1010x; 1.0068x over previous
import functools

import jax
import jax.numpy as jnp
from jax import lax
from jax.experimental import pallas as pl
from jax.experimental.pallas import tpu as pltpu
from jax.experimental.pallas import tpu_sc as plsc

_TOPK = 5
_NPRED = 5000
_NPAD = 5056
_CHUNKS = 316
_NGT = 100
_GTPAD = 104
_B = 8
_NC, _NS = 2, 16
_NW = _NC * _NS
_GT_BLOCKS = _NW // _B
_COLS = _NGT // _GT_BLOCKS


def _sc_match(pb_hbm, gt_hbm, out_hbm,
              px0, py0, px1, py1, pap, g0, g1, g2, g3,
              cbuf, candc, candi, mbc, mbi, outb):
    c = lax.axis_index("c")
    s = lax.axis_index("s")
    w = s * _NC + c
    b = w // _GT_BLOCKS
    gc = w % _GT_BLOCKS

    pltpu.sync_copy(pb_hbm.at[b, 0], px0)
    pltpu.sync_copy(pb_hbm.at[b, 1], py0)
    pltpu.sync_copy(pb_hbm.at[b, 2], px1)
    pltpu.sync_copy(pb_hbm.at[b, 3], py1)
    pltpu.sync_copy(gt_hbm.at[b, 0], g0)
    pltpu.sync_copy(gt_hbm.at[b, 1], g1)
    pltpu.sync_copy(gt_hbm.at[b, 2], g2)
    pltpu.sync_copy(gt_hbm.at[b, 3], g3)

    lane = lax.iota(jnp.int32, 16)
    inf = float("inf")

    def area_body(i, carry):
        o = i * 16
        pap[pl.ds(o, 16)] = ((px1[pl.ds(o, 16)] - px0[pl.ds(o, 16)])
                             * (py1[pl.ds(o, 16)] - py0[pl.ds(o, 16)]))
        return carry

    lax.fori_loop(0, _NPAD // 16, area_body, 0)

    def col_body(col, carry):
        j = gc * _COLS + col
        jv = jnp.full((16,), j, jnp.int32)
        gx0 = plsc.load_gather(g0, [jv])
        gy0 = plsc.load_gather(g1, [jv])
        gx1 = plsc.load_gather(g2, [jv])
        gy1 = plsc.load_gather(g3, [jv])
        area_g = (gx1 - gx0) * (gy1 - gy0)
        inf16 = jnp.full((16,), inf, jnp.float32)

        def cost_of(o):
            x0 = px0[pl.ds(o, 16)]
            y0 = py0[pl.ds(o, 16)]
            x1 = px1[pl.ds(o, 16)]
            y1 = py1[pl.ds(o, 16)]
            ap = pap[pl.ds(o, 16)]
            l1 = (jnp.abs(x0 - gx0) + jnp.abs(y0 - gy0)
                  + jnp.abs(x1 - gx1) + jnp.abs(y1 - gy1))
            ltx = jnp.maximum(x0, gx0)
            lty = jnp.maximum(y0, gy0)
            rbx = jnp.minimum(x1, gx1)
            rby = jnp.minimum(y1, gy1)
            whx = jnp.maximum(rbx - ltx, 0.0)
            why = jnp.maximum(rby - lty, 0.0)
            inter = whx * why
            union = ap + area_g - inter
            iou = inter / union
            cxw = jnp.maximum(x1, gx1) - jnp.minimum(x0, gx0)
            cyw = jnp.maximum(y1, gy1) - jnp.minimum(y0, gy0)
            area_c = cxw * cyw
            giou = iou - (area_c - union) / area_c
            return 1.0 * l1 + 1.0 * (1.0 - giou)

        def pass_a(ch, runmin):
            o = ch * 32
            c0 = cost_of(o)
            c1 = cost_of(o + 16)
            cbuf[pl.ds(o, 16)] = c0
            cbuf[pl.ds(o + 16, 16)] = c1
            return jnp.minimum(runmin, jnp.minimum(c0, c1))

        runmin = lax.fori_loop(0, _CHUNKS // 2, pass_a, inf16)

        smin, _ = plsc.sort_key_val(runmin, lane)
        mbc[pl.ds(0, 16)] = smin
        k4 = jnp.full((16,), _TOPK - 1, jnp.int32)
        tau = plsc.load_gather(mbc, [k4])

        def append(o, cnt):
            cval = cbuf[pl.ds(o, 16)]
            m = cval <= tau
            pref = plsc.cumsum(m.astype(jnp.int32))
            tgt = cnt + pref - 1
            plsc.store_scatter(candc, [tgt], cval, mask=m)
            plsc.store_scatter(candi, [tgt], o + lane, mask=m)
            return cnt + plsc.all_reduce_population_count(m)

        def pass_b(ch, cnt):
            o = ch * 64
            cnt = append(o, cnt)
            cnt = append(o + 16, cnt)
            cnt = append(o + 32, cnt)
            return append(o + 48, cnt)

        cnt = lax.fori_loop(0, _CHUNKS // 4, pass_b,
                            jnp.zeros((16,), jnp.int32))
        cnt_s = jnp.max(cnt)
        ngroups = (cnt_s + 15) // 16

        def pass_c(g, st):
            run_c, run_i = st
            o = g * 16
            ordn = o + lane
            cval = candc[pl.ds(o, 16)]
            ival = candi[pl.ds(o, 16)]
            cval = jnp.where(ordn < cnt, cval, inf)
            skey, sval = plsc.sort_key_val(cval, ival)
            mbc[pl.ds(0, 16)] = run_c
            mbi[pl.ds(0, 16)] = run_i
            mbc[pl.ds(16, 16)] = skey
            mbi[pl.ds(16, 16)] = sval
            gsel = jnp.where(lane < _TOPK, lane, lane + 11)
            mk = plsc.load_gather(mbc, [gsel])
            mv = plsc.load_gather(mbi, [gsel])
            nk, nv = plsc.sort_key_val(mk, mv)
            return (nk, nv)

        run_c, run_i = lax.fori_loop(
            0, ngroups, pass_c, (inf16, jnp.zeros((16,), jnp.int32)))

        plsc.store_scatter(outb, [col * _TOPK + lane], run_i,
                           mask=lane < _TOPK)
        return carry

    lax.fori_loop(0, _COLS, col_body, 0)
    pltpu.sync_copy(outb, out_hbm.at[w])


def kernel(pred_box, pred_obj, gt_box, gt_obj):
    del pred_obj, gt_obj
    B, N, _ = pred_box.shape
    M = gt_box.shape[1]
    poison = jnp.array([1e6, 1e6, 1e6 + 1.0, 1e6 + 1.0],
                       dtype=jnp.float32).reshape(1, 4, 1)
    pb_sc = jnp.concatenate(
        [pred_box.transpose(0, 2, 1),
         jnp.broadcast_to(poison, (B, 4, _NPAD - N))], axis=2)
    gt_sc = jnp.pad(gt_box.transpose(0, 2, 1),
                    ((0, 0), (0, 0), (0, _GTPAD - M)))

    run = pl.kernel(
        _sc_match,
        out_type=jax.ShapeDtypeStruct((_NW, 128), jnp.int32),
        mesh=plsc.VectorSubcoreMesh(core_axis_name="c", subcore_axis_name="s",
                                    num_cores=_NC, num_subcores=_NS),
        scratch_types=(
            [pltpu.VMEM((_NPAD,), jnp.float32)] * 5
            + [pltpu.VMEM((_GTPAD,), jnp.float32)] * 4
            + [pltpu.VMEM((_NPAD,), jnp.float32),
               pltpu.VMEM((_NPAD,), jnp.float32),
               pltpu.VMEM((_NPAD,), jnp.int32),
               pltpu.VMEM((32,), jnp.float32),
               pltpu.VMEM((32,), jnp.int32),
               pltpu.VMEM((128,), jnp.int32)]
        ),
        compiler_params=pltpu.CompilerParams(needs_layout_passes=False),
    )
    out = run(pb_sc, gt_sc)
    matched_pred = out[:, :_COLS * _TOPK].reshape(B, M * _TOPK)
    matched_gt = jnp.broadcast_to(
        jnp.repeat(jnp.arange(M, dtype=jnp.int32), _TOPK), (B, M * _TOPK))
    return matched_pred, matched_gt


# speedup vs baseline: 3.1920x; 1.0293x over previous
import functools

import jax
import jax.numpy as jnp
from jax import lax
from jax.experimental import pallas as pl
from jax.experimental.pallas import tpu as pltpu
from jax.experimental.pallas import tpu_sc as plsc

_TOPK = 5
_NPRED = 5000
_NPAD = 5056
_CHUNKS = 316
_NGT = 100
_GTPAD = 104
_B = 8
_NC, _NS = 2, 16
_NW = _NC * _NS
_GT_BLOCKS = _NW // _B
_COLS = _NGT // _GT_BLOCKS


def _sc_match(pb_hbm, gt_hbm, out_hbm,
              px0, py0, px1, py1, pap, g0, g1, g2, g3,
              cbuf, candi, mbc, mbi, outb):
    c = lax.axis_index("c")
    s = lax.axis_index("s")
    w = s * _NC + c
    b = w // _GT_BLOCKS
    gc = w % _GT_BLOCKS

    pltpu.sync_copy(pb_hbm.at[b, 0], px0)
    pltpu.sync_copy(pb_hbm.at[b, 1], py0)
    pltpu.sync_copy(pb_hbm.at[b, 2], px1)
    pltpu.sync_copy(pb_hbm.at[b, 3], py1)
    pltpu.sync_copy(gt_hbm.at[b, 0], g0)
    pltpu.sync_copy(gt_hbm.at[b, 1], g1)
    pltpu.sync_copy(gt_hbm.at[b, 2], g2)
    pltpu.sync_copy(gt_hbm.at[b, 3], g3)

    lane = lax.iota(jnp.int32, 16)
    inf = float("inf")

    def area_body(i, carry):
        o = i * 16
        pap[pl.ds(o, 16)] = ((px1[pl.ds(o, 16)] - px0[pl.ds(o, 16)])
                             * (py1[pl.ds(o, 16)] - py0[pl.ds(o, 16)]))
        return carry

    lax.fori_loop(0, _NPAD // 16, area_body, 0)

    def col_body(col, carry):
        j = gc * _COLS + col
        jv = jnp.full((16,), j, jnp.int32)
        gx0 = plsc.load_gather(g0, [jv])
        gy0 = plsc.load_gather(g1, [jv])
        gx1 = plsc.load_gather(g2, [jv])
        gy1 = plsc.load_gather(g3, [jv])
        area_g = (gx1 - gx0) * (gy1 - gy0)
        inf16 = jnp.full((16,), inf, jnp.float32)

        def cost_of(o):
            x0 = px0[pl.ds(o, 16)]
            y0 = py0[pl.ds(o, 16)]
            x1 = px1[pl.ds(o, 16)]
            y1 = py1[pl.ds(o, 16)]
            ap = pap[pl.ds(o, 16)]
            l1 = (jnp.abs(x0 - gx0) + jnp.abs(y0 - gy0)
                  + jnp.abs(x1 - gx1) + jnp.abs(y1 - gy1))
            ltx = jnp.maximum(x0, gx0)
            lty = jnp.maximum(y0, gy0)
            rbx = jnp.minimum(x1, gx1)
            rby = jnp.minimum(y1, gy1)
            whx = jnp.maximum(rbx - ltx, 0.0)
            why = jnp.maximum(rby - lty, 0.0)
            inter = whx * why
            union = ap + area_g - inter
            iou = inter / union
            cxw = jnp.maximum(x1, gx1) - jnp.minimum(x0, gx0)
            cyw = jnp.maximum(y1, gy1) - jnp.minimum(y0, gy0)
            area_c = cxw * cyw
            giou = iou - (area_c - union) / area_c
            return 1.0 * l1 + 1.0 * (1.0 - giou)

        def pass_a(ch, runmin):
            o = ch * 64
            c0 = cost_of(o)
            c1 = cost_of(o + 16)
            c2 = cost_of(o + 32)
            c3 = cost_of(o + 48)
            cbuf[pl.ds(o, 16)] = c0
            cbuf[pl.ds(o + 16, 16)] = c1
            cbuf[pl.ds(o + 32, 16)] = c2
            cbuf[pl.ds(o + 48, 16)] = c3
            m01 = jnp.minimum(c0, c1)
            m23 = jnp.minimum(c2, c3)
            return jnp.minimum(runmin, jnp.minimum(m01, m23))

        runmin = lax.fori_loop(0, _CHUNKS // 4, pass_a, inf16)

        smin, _ = plsc.sort_key_val(runmin, lane)
        mbc[pl.ds(0, 16)] = smin
        k4 = jnp.full((16,), _TOPK - 1, jnp.int32)
        tau = plsc.load_gather(mbc, [k4])

        def append(o, cnt):
            cval = cbuf[pl.ds(o, 16)]
            m = cval <= tau
            pref = plsc.cumsum(m.astype(jnp.int32))
            tgt = cnt + pref - 1
            plsc.store_scatter(candi, [tgt], o + lane, mask=m)
            return cnt + plsc.all_reduce_population_count(m)

        def pass_b(ch, cnt):
            o = ch * 64
            cnt = append(o, cnt)
            cnt = append(o + 16, cnt)
            cnt = append(o + 32, cnt)
            return append(o + 48, cnt)

        cnt = lax.fori_loop(0, _CHUNKS // 4, pass_b,
                            jnp.zeros((16,), jnp.int32))
        cnt_s = jnp.max(cnt)
        ngroups = (cnt_s + 15) // 16

        def pass_c(g, st):
            run_c, run_i = st
            o = g * 16
            ordn = o + lane
            valid = ordn < cnt
            ival = jnp.where(valid, candi[pl.ds(o, 16)], 0)
            cval = jnp.where(valid, plsc.load_gather(cbuf, [ival]), inf)
            skey, sval = plsc.sort_key_val(cval, ival)
            mbc[pl.ds(0, 16)] = run_c
            mbi[pl.ds(0, 16)] = run_i
            mbc[pl.ds(16, 16)] = skey
            mbi[pl.ds(16, 16)] = sval
            gsel = jnp.where(lane < _TOPK, lane, lane + 11)
            mk = plsc.load_gather(mbc, [gsel])
            mv = plsc.load_gather(mbi, [gsel])
            nk, nv = plsc.sort_key_val(mk, mv)
            return (nk, nv)

        run_c, run_i = lax.fori_loop(
            0, ngroups, pass_c, (inf16, jnp.zeros((16,), jnp.int32)))

        plsc.store_scatter(outb, [col * _TOPK + lane], run_i,
                           mask=lane < _TOPK)
        return carry

    lax.fori_loop(0, _COLS, col_body, 0)
    pltpu.sync_copy(outb, out_hbm.at[w])


def kernel(pred_box, pred_obj, gt_box, gt_obj):
    del pred_obj, gt_obj
    B, N, _ = pred_box.shape
    M = gt_box.shape[1]
    poison = jnp.array([1e6, 1e6, 1e6 + 1.0, 1e6 + 1.0],
                       dtype=jnp.float32).reshape(1, 4, 1)
    pb_sc = jnp.concatenate(
        [pred_box.transpose(0, 2, 1),
         jnp.broadcast_to(poison, (B, 4, _NPAD - N))], axis=2)
    gt_sc = jnp.pad(gt_box.transpose(0, 2, 1),
                    ((0, 0), (0, 0), (0, _GTPAD - M)))

    run = pl.kernel(
        _sc_match,
        out_type=jax.ShapeDtypeStruct((_NW, 128), jnp.int32),
        mesh=plsc.VectorSubcoreMesh(core_axis_name="c", subcore_axis_name="s",
                                    num_cores=_NC, num_subcores=_NS),
        scratch_types=(
            [pltpu.VMEM((_NPAD,), jnp.float32)] * 5
            + [pltpu.VMEM((_GTPAD,), jnp.float32)] * 4
            + [pltpu.VMEM((_NPAD,), jnp.float32),
               pltpu.VMEM((_NPAD,), jnp.int32),
               pltpu.VMEM((32,), jnp.float32),
               pltpu.VMEM((32,), jnp.int32),
               pltpu.VMEM((128,), jnp.int32)]
        ),
        compiler_params=pltpu.CompilerParams(needs_layout_passes=False),
    )
    out = run(pb_sc, gt_sc)
    matched_pred = out[:, :_COLS * _TOPK].reshape(B, M * _TOPK)
    matched_gt = jnp.broadcast_to(
        jnp.repeat(jnp.arange(M, dtype=jnp.int32), _TOPK), (B, M * _TOPK))
    return matched_pred, matched_gt


# speedup vs baseline: 7.8868x; 2.4708x over previous
import jax
import jax.numpy as jnp
from jax import lax
from jax.experimental import pallas as pl
from jax.experimental.pallas import tpu as pltpu
from jax.experimental.pallas import tpu_sc as plsc

_TOPK = 5
_NPRED = 5000
_NPAD = 5056
_CHUNKS = 316
_NGT = 100
_GTPAD = 104
_B = 8
_NC, _NS = 2, 16
_NW = _NC * _NS
_GT_BLOCKS = _NW // _B
_SC_COLS = 7
_SC_GTS = _GT_BLOCKS * _SC_COLS
_GT_OFF = _NGT - _SC_GTS
_OUTW = 64


def _sc_match(pb_hbm, gt_hbm, out_hbm,
              px0, py0, px1, py1, pap, g0, g1, g2, g3,
              cbuf, candi, mbc, mbi, outb):
    c = lax.axis_index("c")
    s = lax.axis_index("s")
    w = s * _NC + c
    b = w // _GT_BLOCKS
    gc = w % _GT_BLOCKS

    pltpu.sync_copy(pb_hbm.at[b, 0], px0)
    pltpu.sync_copy(pb_hbm.at[b, 1], py0)
    pltpu.sync_copy(pb_hbm.at[b, 2], px1)
    pltpu.sync_copy(pb_hbm.at[b, 3], py1)
    pltpu.sync_copy(gt_hbm.at[b, 0], g0)
    pltpu.sync_copy(gt_hbm.at[b, 1], g1)
    pltpu.sync_copy(gt_hbm.at[b, 2], g2)
    pltpu.sync_copy(gt_hbm.at[b, 3], g3)

    lane = lax.iota(jnp.int32, 16)
    inf = float("inf")

    def area_body(i, carry):
        o = i * 16
        pap[pl.ds(o, 16)] = ((px1[pl.ds(o, 16)] - px0[pl.ds(o, 16)])
                             * (py1[pl.ds(o, 16)] - py0[pl.ds(o, 16)]))
        return carry

    lax.fori_loop(0, _NPAD // 16, area_body, 0)

    def col_body(col, carry):
        j = _GT_OFF + gc * _SC_COLS + col
        jv = jnp.full((16,), j, jnp.int32)
        gx0 = plsc.load_gather(g0, [jv])
        gy0 = plsc.load_gather(g1, [jv])
        gx1 = plsc.load_gather(g2, [jv])
        gy1 = plsc.load_gather(g3, [jv])
        area_g = (gx1 - gx0) * (gy1 - gy0)
        inf16 = jnp.full((16,), inf, jnp.float32)

        def cost_of(o):
            x0 = px0[pl.ds(o, 16)]
            y0 = py0[pl.ds(o, 16)]
            x1 = px1[pl.ds(o, 16)]
            y1 = py1[pl.ds(o, 16)]
            ap = pap[pl.ds(o, 16)]
            l1 = (jnp.abs(x0 - gx0) + jnp.abs(y0 - gy0)
                  + jnp.abs(x1 - gx1) + jnp.abs(y1 - gy1))
            ltx = jnp.maximum(x0, gx0)
            lty = jnp.maximum(y0, gy0)
            rbx = jnp.minimum(x1, gx1)
            rby = jnp.minimum(y1, gy1)
            whx = jnp.maximum(rbx - ltx, 0.0)
            why = jnp.maximum(rby - lty, 0.0)
            inter = whx * why
            union = ap + area_g - inter
            iou = inter / union
            cxw = jnp.maximum(x1, gx1) - jnp.minimum(x0, gx0)
            cyw = jnp.maximum(y1, gy1) - jnp.minimum(y0, gy0)
            area_c = cxw * cyw
            giou = iou - (area_c - union) / area_c
            return 1.0 * l1 + 1.0 * (1.0 - giou)

        def pass_a(ch, runmin):
            o = ch * 64
            c0 = cost_of(o)
            c1 = cost_of(o + 16)
            c2 = cost_of(o + 32)
            c3 = cost_of(o + 48)
            cbuf[pl.ds(o, 16)] = c0
            cbuf[pl.ds(o + 16, 16)] = c1
            cbuf[pl.ds(o + 32, 16)] = c2
            cbuf[pl.ds(o + 48, 16)] = c3
            m01 = jnp.minimum(c0, c1)
            m23 = jnp.minimum(c2, c3)
            return jnp.minimum(runmin, jnp.minimum(m01, m23))

        runmin = lax.fori_loop(0, _CHUNKS // 4, pass_a, inf16)

        smin, _ = plsc.sort_key_val(runmin, lane)
        mbc[pl.ds(0, 16)] = smin
        k4 = jnp.full((16,), _TOPK - 1, jnp.int32)
        tau = plsc.load_gather(mbc, [k4])

        def append(o, cnt):
            cval = cbuf[pl.ds(o, 16)]
            m = cval <= tau
            pref = plsc.cumsum(m.astype(jnp.int32))
            tgt = cnt + pref - 1
            plsc.store_scatter(candi, [tgt], o + lane, mask=m)
            return cnt + plsc.all_reduce_population_count(m)

        def pass_b(ch, cnt):
            o = ch * 64
            cnt = append(o, cnt)
            cnt = append(o + 16, cnt)
            cnt = append(o + 32, cnt)
            return append(o + 48, cnt)

        cnt = lax.fori_loop(0, _CHUNKS // 4, pass_b,
                            jnp.zeros((16,), jnp.int32))
        cnt_s = jnp.max(cnt)
        ngroups = (cnt_s + 15) // 16

        def pass_c(g, st):
            run_c, run_i = st
            o = g * 16
            ordn = o + lane
            valid = ordn < cnt
            ival = jnp.where(valid, candi[pl.ds(o, 16)], 0)
            cval = jnp.where(valid, plsc.load_gather(cbuf, [ival]), inf)
            skey, sval = plsc.sort_key_val(cval, ival)
            mbc[pl.ds(0, 16)] = run_c
            mbi[pl.ds(0, 16)] = run_i
            mbc[pl.ds(16, 16)] = skey
            mbi[pl.ds(16, 16)] = sval
            gsel = jnp.where(lane < _TOPK, lane, lane + 11)
            mk = plsc.load_gather(mbc, [gsel])
            mv = plsc.load_gather(mbi, [gsel])
            nk, nv = plsc.sort_key_val(mk, mv)
            return (nk, nv)

        run_c, run_i = lax.fori_loop(
            0, ngroups, pass_c, (inf16, jnp.zeros((16,), jnp.int32)))

        plsc.store_scatter(outb, [col * _TOPK + lane], run_i,
                           mask=lane < _TOPK)
        return carry

    lax.fori_loop(0, _SC_COLS, col_body, 0)
    pltpu.sync_copy(outb, out_hbm.at[w])


def _tc_match(pb_ref, gt_ref, out_ref):
    pb = pb_ref[0]
    gt = gt_ref[0]
    px0 = pb[0:1, :]
    py0 = pb[1:2, :]
    px1 = pb[2:3, :]
    py1 = pb[3:4, :]
    gx0 = gt[:, 0:1]
    gy0 = gt[:, 1:2]
    gx1 = gt[:, 2:3]
    gy1 = gt[:, 3:4]

    cost_bbox = (jnp.abs(px0 - gx0) + jnp.abs(py0 - gy0)
                 + jnp.abs(px1 - gx1) + jnp.abs(py1 - gy1))
    area_p = (px1 - px0) * (py1 - py0)
    area_g = (gx1 - gx0) * (gy1 - gy0)
    lt_x = jnp.maximum(px0, gx0)
    lt_y = jnp.maximum(py0, gy0)
    rb_x = jnp.minimum(px1, gx1)
    rb_y = jnp.minimum(py1, gy1)
    wh_x = jnp.maximum(rb_x - lt_x, 0.0)
    wh_y = jnp.maximum(rb_y - lt_y, 0.0)
    inter = wh_x * wh_y
    union = area_p + area_g - inter
    iou = inter / union
    cx = jnp.maximum(px1, gx1) - jnp.minimum(px0, gx0)
    cy = jnp.maximum(py1, gy1) - jnp.minimum(py0, gy0)
    area_c = jnp.maximum(cx, 0.0) * jnp.maximum(cy, 0.0)
    giou = iou - (area_c - union) / area_c
    C = 1.0 * cost_bbox + 1.0 * (1.0 - giou)

    n_gt, n_pred = C.shape
    iot = jax.lax.broadcasted_iota(jnp.int32, (n_gt, n_pred), 1)
    cols = []
    for _ in range(_TOPK):
        m = jnp.min(C, axis=1, keepdims=True)
        imin = jnp.min(jnp.where(C == m, iot, 2**30), axis=1,
                       keepdims=True)
        cols.append(imin)
        C = jnp.where(iot == imin, jnp.inf, C)
    out_ref[0] = jnp.concatenate(cols, axis=1)


def kernel(pred_box, pred_obj, gt_box, gt_obj):
    del pred_obj, gt_obj
    B, N, _ = pred_box.shape
    M = gt_box.shape[1]
    pbT = pred_box.transpose(0, 2, 1)
    poison = jnp.array([1e6, 1e6, 1e6 + 1.0, 1e6 + 1.0],
                       dtype=jnp.float32).reshape(1, 4, 1)
    pb_sc = jnp.concatenate(
        [pbT, jnp.broadcast_to(poison, (B, 4, _NPAD - N))], axis=2)
    gt_sc = jnp.pad(gt_box.transpose(0, 2, 1),
                    ((0, 0), (0, 0), (0, _GTPAD - M)))

    run_sc = pl.kernel(
        _sc_match,
        out_type=jax.ShapeDtypeStruct((_NW, _OUTW), jnp.int32),
        mesh=plsc.VectorSubcoreMesh(core_axis_name="c", subcore_axis_name="s",
                                    num_cores=_NC, num_subcores=_NS),
        scratch_types=(
            [pltpu.VMEM((_NPAD,), jnp.float32)] * 5
            + [pltpu.VMEM((_GTPAD,), jnp.float32)] * 4
            + [pltpu.VMEM((_NPAD,), jnp.float32),
               pltpu.VMEM((_NPAD,), jnp.int32),
               pltpu.VMEM((32,), jnp.float32),
               pltpu.VMEM((32,), jnp.int32),
               pltpu.VMEM((_OUTW,), jnp.int32)]
        ),
        compiler_params=pltpu.CompilerParams(needs_layout_passes=False),
    )
    sc_out = run_sc(pb_sc, gt_sc)

    m_tc = _GT_OFF
    tc_idx = pl.pallas_call(
        _tc_match,
        grid=(B,),
        in_specs=[
            pl.BlockSpec((1, 4, N), lambda b: (b, 0, 0)),
            pl.BlockSpec((1, m_tc, 4), lambda b: (b, 0, 0)),
        ],
        out_specs=pl.BlockSpec((1, m_tc, _TOPK), lambda b: (b, 0, 0)),
        out_shape=jax.ShapeDtypeStruct((B, m_tc, _TOPK), jnp.int32),
    )(pbT, gt_box[:, :m_tc, :])

    sc_idx = sc_out[:, :_SC_COLS * _TOPK].reshape(B, _SC_GTS, _TOPK)
    matched_pred = jnp.concatenate([tc_idx, sc_idx], axis=1).reshape(
        B, M * _TOPK)
    matched_gt = jnp.broadcast_to(
        jnp.repeat(jnp.arange(M, dtype=jnp.int32), _TOPK), (B, M * _TOPK))
    return matched_pred, matched_gt


# speedup vs baseline: 8.5747x; 1.0872x over previous
import jax
import jax.numpy as jnp
from jax import lax
from jax.experimental import pallas as pl
from jax.experimental.pallas import tpu as pltpu
from jax.experimental.pallas import tpu_sc as plsc

_TOPK = 5
_NPRED = 5000
_NPAD = 5056
_CHUNKS = 316
_NGT = 100
_GTPAD = 104
_B = 8
_NC, _NS = 2, 16
_NW = _NC * _NS
_GT_BLOCKS = _NW // _B
_SC_COLS = 6
_SC_GTS = _GT_BLOCKS * _SC_COLS
_GT_OFF = _NGT - _SC_GTS
_OUTW = 64


def _sc_match(pb_hbm, gt_hbm, out_hbm,
              px0, py0, px1, py1, pap, g0, g1, g2, g3,
              cbuf, candi, mbc, mbi, outb):
    c = lax.axis_index("c")
    s = lax.axis_index("s")
    w = s * _NC + c
    b = w // _GT_BLOCKS
    gc = w % _GT_BLOCKS

    pltpu.sync_copy(pb_hbm.at[b, 0], px0)
    pltpu.sync_copy(pb_hbm.at[b, 1], py0)
    pltpu.sync_copy(pb_hbm.at[b, 2], px1)
    pltpu.sync_copy(pb_hbm.at[b, 3], py1)
    pltpu.sync_copy(gt_hbm.at[b, 0], g0)
    pltpu.sync_copy(gt_hbm.at[b, 1], g1)
    pltpu.sync_copy(gt_hbm.at[b, 2], g2)
    pltpu.sync_copy(gt_hbm.at[b, 3], g3)

    lane = lax.iota(jnp.int32, 16)
    inf = float("inf")

    def area_body(i, carry):
        o = i * 16
        pap[pl.ds(o, 16)] = ((px1[pl.ds(o, 16)] - px0[pl.ds(o, 16)])
                             * (py1[pl.ds(o, 16)] - py0[pl.ds(o, 16)]))
        return carry

    lax.fori_loop(0, _NPAD // 16, area_body, 0)

    def col_body(col, carry):
        j = _GT_OFF + gc * _SC_COLS + col
        jv = jnp.full((16,), j, jnp.int32)
        gx0 = plsc.load_gather(g0, [jv])
        gy0 = plsc.load_gather(g1, [jv])
        gx1 = plsc.load_gather(g2, [jv])
        gy1 = plsc.load_gather(g3, [jv])
        area_g = (gx1 - gx0) * (gy1 - gy0)
        inf16 = jnp.full((16,), inf, jnp.float32)

        def cost_of(o):
            x0 = px0[pl.ds(o, 16)]
            y0 = py0[pl.ds(o, 16)]
            x1 = px1[pl.ds(o, 16)]
            y1 = py1[pl.ds(o, 16)]
            ap = pap[pl.ds(o, 16)]
            l1 = (jnp.abs(x0 - gx0) + jnp.abs(y0 - gy0)
                  + jnp.abs(x1 - gx1) + jnp.abs(y1 - gy1))
            ltx = jnp.maximum(x0, gx0)
            lty = jnp.maximum(y0, gy0)
            rbx = jnp.minimum(x1, gx1)
            rby = jnp.minimum(y1, gy1)
            whx = jnp.maximum(rbx - ltx, 0.0)
            why = jnp.maximum(rby - lty, 0.0)
            inter = whx * why
            union = ap + area_g - inter
            iou = inter / union
            cxw = jnp.maximum(x1, gx1) - jnp.minimum(x0, gx0)
            cyw = jnp.maximum(y1, gy1) - jnp.minimum(y0, gy0)
            area_c = cxw * cyw
            giou = iou - (area_c - union) / area_c
            return 1.0 * l1 + 1.0 * (1.0 - giou)

        def pass_a(ch, runmin):
            o = ch * 64
            c0 = cost_of(o)
            c1 = cost_of(o + 16)
            c2 = cost_of(o + 32)
            c3 = cost_of(o + 48)
            cbuf[pl.ds(o, 16)] = c0
            cbuf[pl.ds(o + 16, 16)] = c1
            cbuf[pl.ds(o + 32, 16)] = c2
            cbuf[pl.ds(o + 48, 16)] = c3
            m01 = jnp.minimum(c0, c1)
            m23 = jnp.minimum(c2, c3)
            return jnp.minimum(runmin, jnp.minimum(m01, m23))

        runmin = lax.fori_loop(0, _CHUNKS // 4, pass_a, inf16)

        smin, _ = plsc.sort_key_val(runmin, lane)
        mbc[pl.ds(0, 16)] = smin
        k4 = jnp.full((16,), _TOPK - 1, jnp.int32)
        tau = plsc.load_gather(mbc, [k4])

        def append(o, cnt):
            cval = cbuf[pl.ds(o, 16)]
            m = cval <= tau
            pref = plsc.cumsum(m.astype(jnp.int32))
            tgt = cnt + pref - 1
            plsc.store_scatter(candi, [tgt], o + lane, mask=m)
            return cnt + plsc.all_reduce_population_count(m)

        def pass_b(ch, cnt):
            o = ch * 64
            cnt = append(o, cnt)
            cnt = append(o + 16, cnt)
            cnt = append(o + 32, cnt)
            return append(o + 48, cnt)

        cnt = lax.fori_loop(0, _CHUNKS // 4, pass_b,
                            jnp.zeros((16,), jnp.int32))
        cnt_s = jnp.max(cnt)
        ngroups = (cnt_s + 15) // 16

        def pass_c(g, st):
            run_c, run_i = st
            o = g * 16
            ordn = o + lane
            valid = ordn < cnt
            ival = jnp.where(valid, candi[pl.ds(o, 16)], 0)
            cval = jnp.where(valid, plsc.load_gather(cbuf, [ival]), inf)
            skey, sval = plsc.sort_key_val(cval, ival)
            mbc[pl.ds(0, 16)] = run_c
            mbi[pl.ds(0, 16)] = run_i
            mbc[pl.ds(16, 16)] = skey
            mbi[pl.ds(16, 16)] = sval
            gsel = jnp.where(lane < _TOPK, lane, lane + 11)
            mk = plsc.load_gather(mbc, [gsel])
            mv = plsc.load_gather(mbi, [gsel])
            nk, nv = plsc.sort_key_val(mk, mv)
            return (nk, nv)

        run_c, run_i = lax.fori_loop(
            0, ngroups, pass_c, (inf16, jnp.zeros((16,), jnp.int32)))

        plsc.store_scatter(outb, [col * _TOPK + lane], run_i,
                           mask=lane < _TOPK)
        return carry

    lax.fori_loop(0, _SC_COLS, col_body, 0)
    pltpu.sync_copy(outb, out_hbm.at[w])


def _tc_match(pb_ref, gt_ref, out_ref):
    pb = pb_ref[0]
    gt = gt_ref[0]
    px0 = pb[0:1, :_NPRED]
    py0 = pb[1:2, :_NPRED]
    px1 = pb[2:3, :_NPRED]
    py1 = pb[3:4, :_NPRED]
    gx0 = gt[:, 0:1]
    gy0 = gt[:, 1:2]
    gx1 = gt[:, 2:3]
    gy1 = gt[:, 3:4]

    cost_bbox = (jnp.abs(px0 - gx0) + jnp.abs(py0 - gy0)
                 + jnp.abs(px1 - gx1) + jnp.abs(py1 - gy1))
    area_p = (px1 - px0) * (py1 - py0)
    area_g = (gx1 - gx0) * (gy1 - gy0)
    lt_x = jnp.maximum(px0, gx0)
    lt_y = jnp.maximum(py0, gy0)
    rb_x = jnp.minimum(px1, gx1)
    rb_y = jnp.minimum(py1, gy1)
    wh_x = jnp.maximum(rb_x - lt_x, 0.0)
    wh_y = jnp.maximum(rb_y - lt_y, 0.0)
    inter = wh_x * wh_y
    union = area_p + area_g - inter
    iou = inter / union
    cx = jnp.maximum(px1, gx1) - jnp.minimum(px0, gx0)
    cy = jnp.maximum(py1, gy1) - jnp.minimum(py0, gy0)
    area_c = jnp.maximum(cx, 0.0) * jnp.maximum(cy, 0.0)
    giou = iou - (area_c - union) / area_c
    C = 1.0 * cost_bbox + 1.0 * (1.0 - giou)

    n_gt, n_pred = C.shape
    iot = jax.lax.broadcasted_iota(jnp.int32, (n_gt, n_pred), 1)
    cols = []
    for _ in range(_TOPK):
        m = jnp.min(C, axis=1, keepdims=True)
        imin = jnp.min(jnp.where(C == m, iot, 2**30), axis=1,
                       keepdims=True)
        cols.append(imin)
        C = jnp.where(iot == imin, jnp.inf, C)
    out_ref[0] = jnp.concatenate(cols, axis=1)


def kernel(pred_box, pred_obj, gt_box, gt_obj):
    del pred_obj, gt_obj
    B, N, _ = pred_box.shape
    M = gt_box.shape[1]
    pb_sc = jnp.pad(pred_box, ((0, 0), (0, _NPAD - N), (0, 0)),
                    constant_values=1e6).transpose(0, 2, 1)
    gt_sc = jnp.pad(gt_box.transpose(0, 2, 1),
                    ((0, 0), (0, 0), (0, _GTPAD - M)))

    run_sc = pl.kernel(
        _sc_match,
        out_type=jax.ShapeDtypeStruct((_NW, _OUTW), jnp.int32),
        mesh=plsc.VectorSubcoreMesh(core_axis_name="c", subcore_axis_name="s",
                                    num_cores=_NC, num_subcores=_NS),
        scratch_types=(
            [pltpu.VMEM((_NPAD,), jnp.float32)] * 5
            + [pltpu.VMEM((_GTPAD,), jnp.float32)] * 4
            + [pltpu.VMEM((_NPAD,), jnp.float32),
               pltpu.VMEM((_NPAD,), jnp.int32),
               pltpu.VMEM((32,), jnp.float32),
               pltpu.VMEM((32,), jnp.int32),
               pltpu.VMEM((_OUTW,), jnp.int32)]
        ),
        compiler_params=pltpu.CompilerParams(needs_layout_passes=False),
    )
    sc_out = run_sc(pb_sc, gt_sc)

    m_tc = _GT_OFF
    tc_idx = pl.pallas_call(
        _tc_match,
        grid=(B,),
        in_specs=[
            pl.BlockSpec((1, 4, _NPAD), lambda b: (b, 0, 0)),
            pl.BlockSpec((1, m_tc, 4), lambda b: (b, 0, 0)),
        ],
        out_specs=pl.BlockSpec((1, m_tc, _TOPK), lambda b: (b, 0, 0)),
        out_shape=jax.ShapeDtypeStruct((B, m_tc, _TOPK), jnp.int32),
    )(pb_sc, gt_box[:, :m_tc, :])

    sc_idx = sc_out[:, :_SC_COLS * _TOPK].reshape(B, _SC_GTS, _TOPK)
    matched_pred = jnp.concatenate([tc_idx, sc_idx], axis=1).reshape(
        B, M * _TOPK)
    matched_gt = jnp.broadcast_to(
        jnp.repeat(jnp.arange(M, dtype=jnp.int32), _TOPK), (B, M * _TOPK))
    return matched_pred, matched_gt


# speedup vs baseline: 8.6378x; 1.0074x over previous
import jax
import jax.numpy as jnp
from jax import lax
from jax.experimental import pallas as pl
from jax.experimental.pallas import tpu as pltpu
from jax.experimental.pallas import tpu_sc as plsc

_TOPK = 5
_NPRED = 5000
_NPAD = 5056
_CHUNKS = 316
_NGT = 100
_GTPAD = 104
_B = 8
_NC, _NS = 2, 16
_NW = _NC * _NS
_GT_BLOCKS = _NW // _B
_SC_COLS = 5
_SC_GTS = _GT_BLOCKS * _SC_COLS
_GT_OFF = _NGT - _SC_GTS
_OUTW = 64


def _sc_match(pb_hbm, gt_hbm, out_hbm,
              px0, py0, px1, py1, pap, g0, g1, g2, g3,
              cbuf, candi, mbc, mbi, outb):
    c = lax.axis_index("c")
    s = lax.axis_index("s")
    w = s * _NC + c
    b = w // _GT_BLOCKS
    gc = w % _GT_BLOCKS

    pltpu.sync_copy(pb_hbm.at[b, 0], px0)
    pltpu.sync_copy(pb_hbm.at[b, 1], py0)
    pltpu.sync_copy(pb_hbm.at[b, 2], px1)
    pltpu.sync_copy(pb_hbm.at[b, 3], py1)
    pltpu.sync_copy(gt_hbm.at[b, 0], g0)
    pltpu.sync_copy(gt_hbm.at[b, 1], g1)
    pltpu.sync_copy(gt_hbm.at[b, 2], g2)
    pltpu.sync_copy(gt_hbm.at[b, 3], g3)

    lane = lax.iota(jnp.int32, 16)
    inf = float("inf")

    def area_body(i, carry):
        o = i * 16
        pap[pl.ds(o, 16)] = ((px1[pl.ds(o, 16)] - px0[pl.ds(o, 16)])
                             * (py1[pl.ds(o, 16)] - py0[pl.ds(o, 16)]))
        return carry

    lax.fori_loop(0, _NPAD // 16, area_body, 0)

    def col_body(col, carry):
        j = _GT_OFF + gc * _SC_COLS + col
        jv = jnp.full((16,), j, jnp.int32)
        gx0 = plsc.load_gather(g0, [jv])
        gy0 = plsc.load_gather(g1, [jv])
        gx1 = plsc.load_gather(g2, [jv])
        gy1 = plsc.load_gather(g3, [jv])
        area_g = (gx1 - gx0) * (gy1 - gy0)
        inf16 = jnp.full((16,), inf, jnp.float32)

        def cost_of(o):
            x0 = px0[pl.ds(o, 16)]
            y0 = py0[pl.ds(o, 16)]
            x1 = px1[pl.ds(o, 16)]
            y1 = py1[pl.ds(o, 16)]
            ap = pap[pl.ds(o, 16)]
            l1 = (jnp.abs(x0 - gx0) + jnp.abs(y0 - gy0)
                  + jnp.abs(x1 - gx1) + jnp.abs(y1 - gy1))
            ltx = jnp.maximum(x0, gx0)
            lty = jnp.maximum(y0, gy0)
            rbx = jnp.minimum(x1, gx1)
            rby = jnp.minimum(y1, gy1)
            whx = jnp.maximum(rbx - ltx, 0.0)
            why = jnp.maximum(rby - lty, 0.0)
            inter = whx * why
            union = ap + area_g - inter
            iou = inter / union
            cxw = jnp.maximum(x1, gx1) - jnp.minimum(x0, gx0)
            cyw = jnp.maximum(y1, gy1) - jnp.minimum(y0, gy0)
            area_c = cxw * cyw
            giou = iou - (area_c - union) / area_c
            return 1.0 * l1 + 1.0 * (1.0 - giou)

        def pass_a(ch, runmin):
            o = ch * 64
            c0 = cost_of(o)
            c1 = cost_of(o + 16)
            c2 = cost_of(o + 32)
            c3 = cost_of(o + 48)
            cbuf[pl.ds(o, 16)] = c0
            cbuf[pl.ds(o + 16, 16)] = c1
            cbuf[pl.ds(o + 32, 16)] = c2
            cbuf[pl.ds(o + 48, 16)] = c3
            m01 = jnp.minimum(c0, c1)
            m23 = jnp.minimum(c2, c3)
            return jnp.minimum(runmin, jnp.minimum(m01, m23))

        runmin = lax.fori_loop(0, _CHUNKS // 4, pass_a, inf16)

        smin, _ = plsc.sort_key_val(runmin, lane)
        mbc[pl.ds(0, 16)] = smin
        k4 = jnp.full((16,), _TOPK - 1, jnp.int32)
        tau = plsc.load_gather(mbc, [k4])

        def append(o, cnt):
            cval = cbuf[pl.ds(o, 16)]
            m = cval <= tau
            pref = plsc.cumsum(m.astype(jnp.int32))
            tgt = cnt + pref - 1
            plsc.store_scatter(candi, [tgt], o + lane, mask=m)
            return cnt + plsc.all_reduce_population_count(m)

        def pass_b(ch, cnt):
            o = ch * 64
            cnt = append(o, cnt)
            cnt = append(o + 16, cnt)
            cnt = append(o + 32, cnt)
            return append(o + 48, cnt)

        cnt = lax.fori_loop(0, _CHUNKS // 4, pass_b,
                            jnp.zeros((16,), jnp.int32))
        cnt_s = jnp.max(cnt)
        ngroups = (cnt_s + 15) // 16

        def pass_c(g, st):
            run_c, run_i = st
            o = g * 16
            ordn = o + lane
            valid = ordn < cnt
            ival = jnp.where(valid, candi[pl.ds(o, 16)], 0)
            cval = jnp.where(valid, plsc.load_gather(cbuf, [ival]), inf)
            skey, sval = plsc.sort_key_val(cval, ival)
            mbc[pl.ds(0, 16)] = run_c
            mbi[pl.ds(0, 16)] = run_i
            mbc[pl.ds(16, 16)] = skey
            mbi[pl.ds(16, 16)] = sval
            gsel = jnp.where(lane < _TOPK, lane, lane + 11)
            mk = plsc.load_gather(mbc, [gsel])
            mv = plsc.load_gather(mbi, [gsel])
            nk, nv = plsc.sort_key_val(mk, mv)
            return (nk, nv)

        run_c, run_i = lax.fori_loop(
            0, ngroups, pass_c, (inf16, jnp.zeros((16,), jnp.int32)))

        plsc.store_scatter(outb, [col * _TOPK + lane], run_i,
                           mask=lane < _TOPK)
        return carry

    lax.fori_loop(0, _SC_COLS, col_body, 0)
    pltpu.sync_copy(outb, out_hbm.at[w])


def _tc_match(pb_ref, gt_ref, out_ref):
    pb = pb_ref[0]
    gt = gt_ref[0]
    px0 = pb[0:1, :_NPRED]
    py0 = pb[1:2, :_NPRED]
    px1 = pb[2:3, :_NPRED]
    py1 = pb[3:4, :_NPRED]
    gx0 = gt[:, 0:1]
    gy0 = gt[:, 1:2]
    gx1 = gt[:, 2:3]
    gy1 = gt[:, 3:4]

    cost_bbox = (jnp.abs(px0 - gx0) + jnp.abs(py0 - gy0)
                 + jnp.abs(px1 - gx1) + jnp.abs(py1 - gy1))
    area_p = (px1 - px0) * (py1 - py0)
    area_g = (gx1 - gx0) * (gy1 - gy0)
    lt_x = jnp.maximum(px0, gx0)
    lt_y = jnp.maximum(py0, gy0)
    rb_x = jnp.minimum(px1, gx1)
    rb_y = jnp.minimum(py1, gy1)
    wh_x = jnp.maximum(rb_x - lt_x, 0.0)
    wh_y = jnp.maximum(rb_y - lt_y, 0.0)
    inter = wh_x * wh_y
    union = area_p + area_g - inter
    iou = inter / union
    cx = jnp.maximum(px1, gx1) - jnp.minimum(px0, gx0)
    cy = jnp.maximum(py1, gy1) - jnp.minimum(py0, gy0)
    area_c = jnp.maximum(cx, 0.0) * jnp.maximum(cy, 0.0)
    giou = iou - (area_c - union) / area_c
    C = 1.0 * cost_bbox + 1.0 * (1.0 - giou)

    n_gt, n_pred = C.shape
    iot = jax.lax.broadcasted_iota(jnp.int32, (n_gt, n_pred), 1)
    cols = []
    for _ in range(_TOPK):
        m = jnp.min(C, axis=1, keepdims=True)
        imin = jnp.min(jnp.where(C == m, iot, 2**30), axis=1,
                       keepdims=True)
        cols.append(imin)
        C = jnp.where(iot == imin, jnp.inf, C)
    out_ref[0] = jnp.concatenate(cols, axis=1)


def kernel(pred_box, pred_obj, gt_box, gt_obj):
    del pred_obj, gt_obj
    B, N, _ = pred_box.shape
    M = gt_box.shape[1]
    pb_sc = jnp.pad(pred_box, ((0, 0), (0, _NPAD - N), (0, 0)),
                    constant_values=1e6).transpose(0, 2, 1)
    gt_sc = jnp.pad(gt_box.transpose(0, 2, 1),
                    ((0, 0), (0, 0), (0, _GTPAD - M)))

    run_sc = pl.kernel(
        _sc_match,
        out_type=jax.ShapeDtypeStruct((_NW, _OUTW), jnp.int32),
        mesh=plsc.VectorSubcoreMesh(core_axis_name="c", subcore_axis_name="s",
                                    num_cores=_NC, num_subcores=_NS),
        scratch_types=(
            [pltpu.VMEM((_NPAD,), jnp.float32)] * 5
            + [pltpu.VMEM((_GTPAD,), jnp.float32)] * 4
            + [pltpu.VMEM((_NPAD,), jnp.float32),
               pltpu.VMEM((_NPAD,), jnp.int32),
               pltpu.VMEM((32,), jnp.float32),
               pltpu.VMEM((32,), jnp.int32),
               pltpu.VMEM((_OUTW,), jnp.int32)]
        ),
        compiler_params=pltpu.CompilerParams(needs_layout_passes=False),
    )
    sc_out = run_sc(pb_sc, gt_sc)

    m_tc = _GT_OFF
    tc_idx = pl.pallas_call(
        _tc_match,
        grid=(B,),
        in_specs=[
            pl.BlockSpec((1, 4, _NPAD), lambda b: (b, 0, 0)),
            pl.BlockSpec((1, m_tc, 4), lambda b: (b, 0, 0)),
        ],
        out_specs=pl.BlockSpec((1, m_tc, _TOPK), lambda b: (b, 0, 0)),
        out_shape=jax.ShapeDtypeStruct((B, m_tc, _TOPK), jnp.int32),
    )(pb_sc, gt_box[:, :m_tc, :])

    sc_idx = sc_out[:, :_SC_COLS * _TOPK].reshape(B, _SC_GTS, _TOPK)
    matched_pred = jnp.concatenate([tc_idx, sc_idx], axis=1).reshape(
        B, M * _TOPK)
    matched_gt = jnp.broadcast_to(
        jnp.repeat(jnp.arange(M, dtype=jnp.int32), _TOPK), (B, M * _TOPK))
    return matched_pred, matched_gt
